# Initial kernel scaffold; baseline (speedup 1.0000x reference)
import jax
import jax.numpy as jnp
from jax.experimental import pallas as pl


def kernel(input_ids, table, fc_w, fc_b):
    raise NotImplementedError("write your pallas kernel here")


# speedup vs baseline: 2.1074x; 2.1074x over previous
import functools

import jax
import jax.numpy as jnp
from jax import lax
from jax.experimental import pallas as pl
from jax.experimental.pallas import tpu as pltpu
from jax.experimental.pallas import tpu_sc as plsc

D = 128
HIST = 200
B = 4096
OUT = 512

_NC, _NS = 2, 16
NW = _NC * _NS
BPW = B // NW
LANES = 16
DB = D // LANES

CHUNK0 = 128
CHUNK1 = HIST - CHUNK0


def _sc_pool_body(ids_hbm, table_hbm, pooled_hbm, idx_v, rows_v, out_v,
                  sem0, sem1):
    wid = lax.axis_index("s") * _NC + lax.axis_index("c")
    base = wid * BPW
    pltpu.sync_copy(ids_hbm.at[pl.ds(base * HIST, BPW * HIST)], idx_v)

    sems = (sem0, sem1)

    def issue(b, buf):
        off = b * HIST
        pltpu.async_copy(table_hbm.at[idx_v.at[pl.ds(off, CHUNK0)]],
                         rows_v.at[buf, pl.ds(0, CHUNK0)], sems[buf])
        pltpu.async_copy(table_hbm.at[idx_v.at[pl.ds(off + CHUNK0, CHUNK1)]],
                         rows_v.at[buf, pl.ds(CHUNK0, CHUNK1)], sems[buf])

    def wait(buf):
        pltpu.make_async_copy(table_hbm.at[pl.ds(0, HIST)], rows_v.at[buf],
                              sems[buf]).wait()

    def accum(b, buf):
        def acc_body(r, carry):
            return tuple(carry[k] + rows_v[buf, r, pl.ds(k * LANES, LANES)]
                         for k in range(DB))
        init = tuple(jnp.zeros((LANES,), jnp.float32) for _ in range(DB))
        acc = lax.fori_loop(0, HIST, acc_body, init)
        for k in range(DB):
            out_v[b, pl.ds(k * LANES, LANES)] = acc[k]

    issue(0, 0)

    def loop_body(i, carry):
        b0 = 2 * i
        issue(b0 + 1, 1)
        wait(0)
        accum(b0, 0)

        @pl.when(b0 + 2 < BPW)
        def _():
            issue(b0 + 2, 0)

        wait(1)
        accum(b0 + 1, 1)
        return carry

    lax.fori_loop(0, BPW // 2, loop_body, 0)
    pltpu.sync_copy(out_v, pooled_hbm.at[pl.ds(base, BPW)])


@functools.cache
def _sc_pool():
    return pl.kernel(
        _sc_pool_body,
        out_type=jax.ShapeDtypeStruct((B, D), jnp.float32),
        mesh=plsc.VectorSubcoreMesh(core_axis_name="c", subcore_axis_name="s",
                                    num_cores=_NC, num_subcores=_NS),
        scratch_types=[
            pltpu.VMEM((BPW * HIST,), jnp.int32),
            pltpu.VMEM((2, HIST, D), jnp.float32),
            pltpu.VMEM((BPW, D), jnp.float32),
            pltpu.SemaphoreType.DMA,
            pltpu.SemaphoreType.DMA,
        ],
    )


def _tc_fc_body(pooled_ref, w_ref, b_ref, out_ref):
    x = pooled_ref[...] * (1.0 / HIST)
    out_ref[...] = (
        jnp.dot(x, w_ref[...], preferred_element_type=jnp.float32,
                precision=lax.Precision.HIGHEST)
        + b_ref[...]
    )


_BM = 256


def _tc_fc(pooled, fc_w, fc_b2):
    return pl.pallas_call(
        _tc_fc_body,
        out_shape=jax.ShapeDtypeStruct((B, OUT), jnp.float32),
        grid=(B // _BM,),
        in_specs=[
            pl.BlockSpec((_BM, D), lambda i: (i, 0)),
            pl.BlockSpec((D, OUT), lambda i: (0, 0)),
            pl.BlockSpec((1, OUT), lambda i: (0, 0)),
        ],
        out_specs=pl.BlockSpec((_BM, OUT), lambda i: (i, 0)),
    )(pooled, fc_w, fc_b2)


def kernel(input_ids, table, fc_w, fc_b):
    ids_flat = input_ids.reshape(-1).astype(jnp.int32)
    pooled = _sc_pool()(ids_flat, table)
    return _tc_fc(pooled, fc_w, fc_b.reshape(1, OUT))


# speedup vs baseline: 2.1100x; 1.0012x over previous
import functools

import jax
import jax.numpy as jnp
from jax import lax
from jax.experimental import pallas as pl
from jax.experimental.pallas import tpu as pltpu
from jax.experimental.pallas import tpu_sc as plsc

D = 128
HIST = 200
B = 4096
OUT = 512

_NC, _NS = 2, 16
NW = _NC * _NS
BPW = B // NW
LANES = 16
DB = D // LANES

CHUNK0 = 128
CHUNK1 = HIST - CHUNK0


def _sc_pool_body(ids_hbm, table_hbm, pooled_hbm, idx_v, rows_v, out_v,
                  sem0, sem1):
    wid = lax.axis_index("s") * _NC + lax.axis_index("c")
    base = wid * BPW
    pltpu.sync_copy(ids_hbm.at[pl.ds(base * HIST, BPW * HIST)], idx_v)

    sems = (sem0, sem1)

    def issue(b, buf):
        off = b * HIST
        pltpu.async_copy(table_hbm.at[idx_v.at[pl.ds(off, CHUNK0)]],
                         rows_v.at[buf, pl.ds(0, CHUNK0)], sems[buf])
        pltpu.async_copy(table_hbm.at[idx_v.at[pl.ds(off + CHUNK0, CHUNK1)]],
                         rows_v.at[buf, pl.ds(CHUNK0, CHUNK1)], sems[buf])

    def wait(buf):
        pltpu.make_async_copy(table_hbm.at[pl.ds(0, HIST)], rows_v.at[buf],
                              sems[buf]).wait()

    def accum(b, buf):
        def acc_body(r, carry):
            return tuple(carry[k] + rows_v[buf, r, pl.ds(k * LANES, LANES)]
                         for k in range(DB))
        init = tuple(jnp.zeros((LANES,), jnp.float32) for _ in range(DB))
        acc = lax.fori_loop(0, HIST, acc_body, init, unroll=8)
        for k in range(DB):
            out_v[b, pl.ds(k * LANES, LANES)] = acc[k]

    issue(0, 0)

    def loop_body(i, carry):
        b0 = 2 * i
        issue(b0 + 1, 1)
        wait(0)
        accum(b0, 0)

        @pl.when(b0 + 2 < BPW)
        def _():
            issue(b0 + 2, 0)

        wait(1)
        accum(b0 + 1, 1)
        return carry

    lax.fori_loop(0, BPW // 2, loop_body, 0)
    pltpu.sync_copy(out_v, pooled_hbm.at[pl.ds(base, BPW)])


@functools.cache
def _sc_pool():
    return pl.kernel(
        _sc_pool_body,
        out_type=jax.ShapeDtypeStruct((B, D), jnp.float32),
        mesh=plsc.VectorSubcoreMesh(core_axis_name="c", subcore_axis_name="s",
                                    num_cores=_NC, num_subcores=_NS),
        scratch_types=[
            pltpu.VMEM((BPW * HIST,), jnp.int32),
            pltpu.VMEM((2, HIST, D), jnp.float32),
            pltpu.VMEM((BPW, D), jnp.float32),
            pltpu.SemaphoreType.DMA,
            pltpu.SemaphoreType.DMA,
        ],
    )


def _tc_fc_body(pooled_ref, w_ref, b_ref, out_ref):
    x = pooled_ref[...] * (1.0 / HIST)
    out_ref[...] = (
        jnp.dot(x, w_ref[...], preferred_element_type=jnp.float32,
                precision=lax.Precision.HIGHEST)
        + b_ref[...]
    )


_BM = 256


def _tc_fc(pooled, fc_w, fc_b2):
    return pl.pallas_call(
        _tc_fc_body,
        out_shape=jax.ShapeDtypeStruct((B, OUT), jnp.float32),
        grid=(B // _BM,),
        in_specs=[
            pl.BlockSpec((_BM, D), lambda i: (i, 0)),
            pl.BlockSpec((D, OUT), lambda i: (0, 0)),
            pl.BlockSpec((1, OUT), lambda i: (0, 0)),
        ],
        out_specs=pl.BlockSpec((_BM, OUT), lambda i: (i, 0)),
    )(pooled, fc_w, fc_b2)


def kernel(input_ids, table, fc_w, fc_b):
    ids_flat = input_ids.reshape(-1).astype(jnp.int32)
    pooled = _sc_pool()(ids_flat, table)
    return _tc_fc(pooled, fc_w, fc_b.reshape(1, OUT))


# speedup vs baseline: 2.5358x; 1.2018x over previous
import functools

import jax
import jax.numpy as jnp
from jax import lax
from jax.experimental import pallas as pl
from jax.experimental.pallas import tpu as pltpu
from jax.experimental.pallas import tpu_sc as plsc

D = 128
HIST = 200
B = 4096
OUT = 512

_NC, _NS = 2, 16
NW = _NC * _NS
BPW = B // NW
LANES = 16
DB = D // LANES

CHUNK0 = 128
CHUNK1 = HIST - CHUNK0


NBUF = 3


def _sc_pool_body(ids_hbm, table_hbm, pooled_hbm, idx_v, rows_v, out_v,
                  sem0, sem1, sem2):
    wid = lax.axis_index("s") * _NC + lax.axis_index("c")
    base = wid * BPW
    pltpu.sync_copy(ids_hbm.at[pl.ds(base * HIST, BPW * HIST)], idx_v)

    sems = (sem0, sem1, sem2)

    def issue(b, buf):
        off = b * HIST
        pltpu.async_copy(table_hbm.at[idx_v.at[pl.ds(off, CHUNK0)]],
                         rows_v.at[buf, pl.ds(0, CHUNK0)], sems[buf])
        pltpu.async_copy(table_hbm.at[idx_v.at[pl.ds(off + CHUNK0, CHUNK1)]],
                         rows_v.at[buf, pl.ds(CHUNK0, CHUNK1)], sems[buf])

    def wait(buf):
        pltpu.make_async_copy(table_hbm.at[pl.ds(0, HIST)], rows_v.at[buf],
                              sems[buf]).wait()

    def accum(b, buf):
        def acc_body(r, carry):
            return tuple(carry[k] + rows_v[buf, r, pl.ds(k * LANES, LANES)]
                         for k in range(DB))
        init = tuple(jnp.zeros((LANES,), jnp.float32) for _ in range(DB))
        acc = lax.fori_loop(0, HIST, acc_body, init, unroll=8)
        for k in range(DB):
            out_v[b, pl.ds(k * LANES, LANES)] = acc[k]

    for j in range(NBUF):
        issue(j, j)

    NFULL = (BPW - NBUF) // NBUF

    def loop_body(i, carry):
        for j in range(NBUF):
            b = NBUF * i + j
            wait(j)
            accum(b, j)
            issue(b + NBUF, j)
        return carry

    lax.fori_loop(0, NFULL, loop_body, 0)
    for b in range(NFULL * NBUF, BPW):
        wait(b % NBUF)
        accum(b, b % NBUF)
        if b + NBUF < BPW:
            issue(b + NBUF, b % NBUF)
    pltpu.sync_copy(out_v, pooled_hbm.at[pl.ds(base, BPW)])


@functools.cache
def _sc_pool():
    return pl.kernel(
        _sc_pool_body,
        out_type=jax.ShapeDtypeStruct((B, D), jnp.float32),
        mesh=plsc.VectorSubcoreMesh(core_axis_name="c", subcore_axis_name="s",
                                    num_cores=_NC, num_subcores=_NS),
        scratch_types=[
            pltpu.VMEM((BPW * HIST,), jnp.int32),
            pltpu.VMEM((NBUF, HIST, D), jnp.float32),
            pltpu.VMEM((BPW, D), jnp.float32),
            pltpu.SemaphoreType.DMA,
            pltpu.SemaphoreType.DMA,
            pltpu.SemaphoreType.DMA,
        ],
    )


def _tc_fc_body(pooled_ref, w_ref, b_ref, out_ref):
    x = pooled_ref[...] * (1.0 / HIST)
    out_ref[...] = (
        jnp.dot(x, w_ref[...], preferred_element_type=jnp.float32,
                precision=lax.Precision.HIGHEST)
        + b_ref[...]
    )


_BM = 256


def _tc_fc(pooled, fc_w, fc_b2):
    return pl.pallas_call(
        _tc_fc_body,
        out_shape=jax.ShapeDtypeStruct((B, OUT), jnp.float32),
        grid=(B // _BM,),
        in_specs=[
            pl.BlockSpec((_BM, D), lambda i: (i, 0)),
            pl.BlockSpec((D, OUT), lambda i: (0, 0)),
            pl.BlockSpec((1, OUT), lambda i: (0, 0)),
        ],
        out_specs=pl.BlockSpec((_BM, OUT), lambda i: (i, 0)),
    )(pooled, fc_w, fc_b2)


def kernel(input_ids, table, fc_w, fc_b):
    ids_flat = input_ids.reshape(-1).astype(jnp.int32)
    pooled = _sc_pool()(ids_flat, table)
    return _tc_fc(pooled, fc_w, fc_b.reshape(1, OUT))
